# initial kernel scaffold (unmeasured)
import jax
import jax.numpy as jnp
from jax import lax
from jax.experimental import pallas as pl
from jax.experimental.pallas import tpu as pltpu

N_DEV = 4
M, N = 4096, 8192
MC = M // N_DEV
NC = 1024
HALF = N // 2
T = HALF // NC
STEPS = 2 * (N_DEV - 1)
MSGS = T * STEPS


def kernel(x, w_mat, scale_x, scale_w):
    partial = lax.dot_general(
        x, w_mat, (((1,), (0,)), ((), ())),
        preferred_element_type=jnp.int32,
    )
    scale = (scale_x * scale_w).reshape(1, 1)

    def body(partial_ref, scale_ref, out_ref,
             send_buf, recv_buf, acc,
             send_sems, recv_sems, load_sems, store_sems, own_sems):
        my = lax.axis_index("i")
        sgn = (1, -1)
        nbo = (jnp.mod(my + 1, N_DEV), jnp.mod(my + N_DEV - 1, N_DEV))

        def cidx(d, j):
            return jnp.mod(my - sgn[d] * j + 4 * N_DEV, N_DEV)

        def col0(d, t):
            return d * HALF + t * NC

        def load_chunk(d, c, col):
            cp = pltpu.make_async_copy(
                partial_ref.at[pl.ds(c * MC, MC), pl.ds(col, NC)],
                acc.at[d],
                load_sems.at[d],
            )
            cp.start()
            cp.wait()

        def rdma(d, k):
            return pltpu.make_async_remote_copy(
                src_ref=send_buf.at[d, k % 2],
                dst_ref=recv_buf.at[d, k % 4],
                send_sem=send_sems.at[d, k % 2],
                recv_sem=recv_sems.at[d, k % 4],
                device_id=(nbo[d],),
                device_id_type=pl.DeviceIdType.MESH,
            )

        barrier = pltpu.get_barrier_semaphore()
        for d in range(2):
            pl.semaphore_signal(
                barrier, inc=1, device_id=(nbo[d],),
                device_id_type=pl.DeviceIdType.MESH,
            )
        pl.semaphore_wait(barrier, 2)

        for d in range(2):
            load_chunk(d, my, col0(d, 0))
            send_buf[d, 0] = acc[d].astype(jnp.float32)

        rdmas = {}
        pending_store = [None, None]
        pending_own = [None, None]
        for k in range(MSGS):
            for d in range(2):
                if pending_store[d] is not None:
                    pending_store[d].wait()
                    pending_store[d] = None
            for d in range(2):
                r = rdma(d, k)
                rdmas[(d, k)] = r
                r.start()
            t, s = divmod(k, STEPS)
            for d in range(2):
                col = col0(d, t)
                rdmas[(d, k)].wait_recv()
                if s < 2:
                    c = cidx(d, s + 1)
                    load_chunk(d, c, col)
                    if k >= 1:
                        rdmas[(d, k - 1)].wait_send()
                    send_buf[d, (k + 1) % 2] = (
                        recv_buf[d, k % 4] + acc[d].astype(jnp.float32)
                    )
                elif s == 2:
                    c = cidx(d, 3)
                    load_chunk(d, c, col)
                    rdmas[(d, k - 1)].wait_send()
                    owned = (
                        recv_buf[d, k % 4] + acc[d].astype(jnp.float32)
                    ) * scale_ref[0, 0]
                    send_buf[d, (k + 1) % 2] = jnp.maximum(owned, 0.0)
                    ost = pltpu.make_async_copy(
                        send_buf.at[d, (k + 1) % 2],
                        out_ref.at[pl.ds(c * MC, MC), pl.ds(col, NC)],
                        own_sems.at[d],
                    )
                    ost.start()
                    pending_own[d] = ost
                else:
                    c = cidx(d, s - 3)
                    st = pltpu.make_async_copy(
                        recv_buf.at[d, k % 4],
                        out_ref.at[pl.ds(c * MC, MC), pl.ds(col, NC)],
                        store_sems.at[d],
                    )
                    st.start()
                    pending_store[d] = st
                    if s < STEPS - 1:
                        rdmas[(d, k - 1)].wait_send()
                        if s == 4:
                            pending_own[d].wait()
                            pending_own[d] = None
                        send_buf[d, (k + 1) % 2] = recv_buf[d, k % 4]
                    elif t + 1 < T:
                        rdmas[(d, k - 1)].wait_send()
                        load_chunk(d, my, col0(d, t + 1))
                        send_buf[d, 0] = acc[d].astype(jnp.float32)

        for d in range(2):
            rdmas[(d, MSGS - 2)].wait_send()
            rdmas[(d, MSGS - 1)].wait_send()
            if pending_store[d] is not None:
                pending_store[d].wait()

    return pl.pallas_call(
        body,
        out_shape=jax.ShapeDtypeStruct((M, N), jnp.float32),
        in_specs=[
            pl.BlockSpec(memory_space=pltpu.ANY),
            pl.BlockSpec(memory_space=pltpu.SMEM),
        ],
        out_specs=pl.BlockSpec(memory_space=pltpu.ANY),
        scratch_shapes=[
            pltpu.VMEM((2, 2, MC, NC), jnp.float32),
            pltpu.VMEM((2, 4, MC, NC), jnp.float32),
            pltpu.VMEM((2, MC, NC), jnp.int32),
            pltpu.SemaphoreType.DMA((2, 2)),
            pltpu.SemaphoreType.DMA((2, 4)),
            pltpu.SemaphoreType.DMA((2,)),
            pltpu.SemaphoreType.DMA((2,)),
            pltpu.SemaphoreType.DMA((2,)),
        ],
        compiler_params=pltpu.CompilerParams(collective_id=0),
    )(partial, scale)


# baseline (device time: 1421514 ns/iter reference)
import jax
import jax.numpy as jnp
from jax import lax
from jax.experimental import pallas as pl
from jax.experimental.pallas import tpu as pltpu

N_DEV = 4
M, N = 4096, 8192
MC = M // N_DEV
NC = 1024
HALF = N // 2
T = HALF // NC
STEPS = 2 * (N_DEV - 1)
MSGS = T * STEPS


def kernel(x, w_mat, scale_x, scale_w):
    partial = lax.dot_general(
        x, w_mat, (((1,), (0,)), ((), ())),
        preferred_element_type=jnp.int32,
    )
    scale = (scale_x * scale_w).reshape(1, 1)

    def body(partial_ref, scale_ref, out_ref,
             send_buf, recv_buf, acc,
             send_sems, recv_sems, load_sems, store_sems, own_sems):
        my = lax.axis_index("i")
        sgn = (1, -1)
        nbo = (jnp.mod(my + 1, N_DEV), jnp.mod(my + N_DEV - 1, N_DEV))

        def cidx(d, j):
            return jnp.mod(my - sgn[d] * j + 4 * N_DEV, N_DEV)

        def col0(d, t):
            return d * HALF + t * NC

        def load_chunk(d, c, col):
            cp = pltpu.make_async_copy(
                partial_ref.at[pl.ds(c * MC, MC), pl.ds(col, NC)],
                acc.at[d],
                load_sems.at[d],
            )
            cp.start()
            cp.wait()

        def rdma(d, k):
            return pltpu.make_async_remote_copy(
                src_ref=send_buf.at[d, k % 2],
                dst_ref=recv_buf.at[d, k % 4],
                send_sem=send_sems.at[d, k % 2],
                recv_sem=recv_sems.at[d, k % 4],
                device_id=(nbo[d],),
                device_id_type=pl.DeviceIdType.MESH,
            )

        barrier = pltpu.get_barrier_semaphore()
        for d in range(2):
            pl.semaphore_signal(
                barrier, inc=1, device_id=(nbo[d],),
                device_id_type=pl.DeviceIdType.MESH,
            )
        pl.semaphore_wait(barrier, 2)

        for d in range(2):
            load_chunk(d, my, col0(d, 0))
            send_buf[d, 0] = acc[d].astype(jnp.float32)

        rdmas = {}
        pending_store = [None, None]
        pending_own = [None, None]
        for k in range(MSGS):
            for d in range(2):
                if pending_store[d] is not None:
                    pending_store[d].wait()
                    pending_store[d] = None
            for d in range(2):
                r = rdma(d, k)
                rdmas[(d, k)] = r
                r.start()
            t, s = divmod(k, STEPS)
            for d in range(2):
                col = col0(d, t)
                rdmas[(d, k)].wait_recv()
                if s < 2:
                    c = cidx(d, s + 1)
                    load_chunk(d, c, col)
                    if k >= 1:
                        rdmas[(d, k - 1)].wait_send()
                    send_buf[d, (k + 1) % 2] = (
                        recv_buf[d, k % 4] + acc[d].astype(jnp.float32)
                    )
                elif s == 2:
                    c = cidx(d, 3)
                    load_chunk(d, c, col)
                    rdmas[(d, k - 1)].wait_send()
                    owned = (
                        recv_buf[d, k % 4] + acc[d].astype(jnp.float32)
                    ) * scale_ref[0, 0]
                    send_buf[d, (k + 1) % 2] = jnp.maximum(owned, 0.0)
                    ost = pltpu.make_async_copy(
                        send_buf.at[d, (k + 1) % 2],
                        out_ref.at[pl.ds(c * MC, MC), pl.ds(col, NC)],
                        own_sems.at[d],
                    )
                    ost.start()
                    pending_own[d] = ost
                else:
                    c = cidx(d, s - 3)
                    st = pltpu.make_async_copy(
                        recv_buf.at[d, k % 4],
                        out_ref.at[pl.ds(c * MC, MC), pl.ds(col, NC)],
                        store_sems.at[d],
                    )
                    st.start()
                    pending_store[d] = st
                    if s < STEPS - 1:
                        rdmas[(d, k - 1)].wait_send()
                        if s == 4:
                            pending_own[d].wait()
                            pending_own[d] = None
                        send_buf[d, (k + 1) % 2] = recv_buf[d, k % 4]
                    elif t + 1 < T:
                        rdmas[(d, k - 1)].wait_send()
                        load_chunk(d, my, col0(d, t + 1))
                        send_buf[d, 0] = acc[d].astype(jnp.float32)

        for d in range(2):
            rdmas[(d, MSGS - 2)].wait_send()
            rdmas[(d, MSGS - 1)].wait_send()
            if pending_store[d] is not None:
                pending_store[d].wait()

    return pl.pallas_call(
        body,
        out_shape=jax.ShapeDtypeStruct((M, N), jnp.float32),
        in_specs=[
            pl.BlockSpec(memory_space=pl.ANY),
            pl.BlockSpec(memory_space=pltpu.SMEM),
        ],
        out_specs=pl.BlockSpec(memory_space=pl.ANY),
        scratch_shapes=[
            pltpu.VMEM((2, 2, MC, NC), jnp.float32),
            pltpu.VMEM((2, 4, MC, NC), jnp.float32),
            pltpu.VMEM((2, MC, NC), jnp.int32),
            pltpu.SemaphoreType.DMA((2, 2)),
            pltpu.SemaphoreType.DMA((2, 4)),
            pltpu.SemaphoreType.DMA((2,)),
            pltpu.SemaphoreType.DMA((2,)),
            pltpu.SemaphoreType.DMA((2,)),
        ],
        compiler_params=pltpu.CompilerParams(
            collective_id=0,
            vmem_limit_bytes=100 * 1024 * 1024,
        ),
    )(partial, scale)


# device time: 1261167 ns/iter; 1.1271x vs baseline; 1.1271x over previous
import jax
import jax.numpy as jnp
from jax import lax
from jax.experimental import pallas as pl
from jax.experimental.pallas import tpu as pltpu

N_DEV = 4
M, N = 4096, 8192
MC = M // N_DEV
NC = 512
HALF = N // 2
QUARTER = HALF // 2
T = QUARTER // NC
STEPS = 2 * (N_DEV - 1)
MSGS = T * STEPS


def kernel(x, w_mat, scale_x, scale_w):
    partial = lax.dot_general(
        x, w_mat, (((1,), (0,)), ((), ())),
        preferred_element_type=jnp.int32,
    )
    scale = (scale_x * scale_w).reshape(1, 1)

    def body(partial_ref, scale_ref, out_ref,
             send_buf, recv_buf, acc,
             send_sems, recv_sems, load_sems, store_sems, own_sems):
        my = lax.axis_index("i")
        sgn = (1, -1)
        nbo = (jnp.mod(my + 1, N_DEV), jnp.mod(my + N_DEV - 1, N_DEV))

        def cidx(d, j):
            return jnp.mod(my - sgn[d] * j + 4 * N_DEV, N_DEV)

        def col0(d, X, t):
            return d * HALF + X * QUARTER + t * NC

        loads = {}

        def start_load(d, X, c, col):
            cp = pltpu.make_async_copy(
                partial_ref.at[pl.ds(c * MC, MC), pl.ds(col, NC)],
                acc.at[d, X],
                load_sems.at[d, X],
            )
            cp.start()
            loads[(d, X)] = cp

        def rdma(d, X, k):
            return pltpu.make_async_remote_copy(
                src_ref=send_buf.at[d, X, k % 2],
                dst_ref=recv_buf.at[d, X, k % 4],
                send_sem=send_sems.at[d, X, k % 2],
                recv_sem=recv_sems.at[d, X, k % 4],
                device_id=(nbo[d],),
                device_id_type=pl.DeviceIdType.MESH,
            )

        barrier = pltpu.get_barrier_semaphore()
        for d in range(2):
            pl.semaphore_signal(
                barrier, inc=1, device_id=(nbo[d],),
                device_id_type=pl.DeviceIdType.MESH,
            )
        pl.semaphore_wait(barrier, 2)

        for d in range(2):
            for X in range(2):
                start_load(d, X, my, col0(d, X, 0))
        for d in range(2):
            for X in range(2):
                loads[(d, X)].wait()
                send_buf[d, X, 0] = acc[d, X].astype(jnp.float32)

        rdmas = {}
        pending_store = {}
        pending_own = {}

        def process(d, X, kk):
            t, s = divmod(kk, STEPS)
            col = col0(d, X, t)
            rdmas[(d, X, kk)].wait_recv()
            if s < 2:
                loads[(d, X)].wait()
                rdmas[(d, X, kk - 1)].wait_send() if kk >= 1 else None
                send_buf[d, X, (kk + 1) % 2] = (
                    recv_buf[d, X, kk % 4] + acc[d, X].astype(jnp.float32)
                )
            elif s == 2:
                c = cidx(d, 3)
                loads[(d, X)].wait()
                rdmas[(d, X, kk - 1)].wait_send()
                owned = (
                    recv_buf[d, X, kk % 4] + acc[d, X].astype(jnp.float32)
                ) * scale_ref[0, 0]
                send_buf[d, X, (kk + 1) % 2] = jnp.maximum(owned, 0.0)
                ost = pltpu.make_async_copy(
                    send_buf.at[d, X, (kk + 1) % 2],
                    out_ref.at[pl.ds(c * MC, MC), pl.ds(col, NC)],
                    own_sems.at[d, X],
                )
                ost.start()
                pending_own[(d, X)] = ost
            else:
                c = cidx(d, s - 3)
                st = pltpu.make_async_copy(
                    recv_buf.at[d, X, kk % 4],
                    out_ref.at[pl.ds(c * MC, MC), pl.ds(col, NC)],
                    store_sems.at[d, X],
                )
                st.start()
                pending_store[(d, X)] = st
                if s < STEPS - 1:
                    rdmas[(d, X, kk - 1)].wait_send()
                    if s == 4:
                        pending_own.pop((d, X)).wait()
                    send_buf[d, X, (kk + 1) % 2] = recv_buf[d, X, kk % 4]
                elif t + 1 < T:
                    rdmas[(d, X, kk - 1)].wait_send()
                    loads[(d, X)].wait()
                    send_buf[d, X, 0] = acc[d, X].astype(jnp.float32)

        J = 2 * MSGS
        for j in range(J):
            X, kk = j % 2, j // 2
            t, s = divmod(kk, STEPS)
            for d in range(2):
                st = pending_store.pop((d, X), None)
                if st is not None:
                    st.wait()
                r = rdma(d, X, kk)
                rdmas[(d, X, kk)] = r
                r.start()
                if s < 3:
                    start_load(d, X, cidx(d, s + 1), col0(d, X, t))
                elif s == STEPS - 1 and t + 1 < T:
                    start_load(d, X, my, col0(d, X, t + 1))
            if j >= 1:
                Xp, kp = (j - 1) % 2, (j - 1) // 2
                for d in range(2):
                    process(d, Xp, kp)
        for d in range(2):
            process(d, (J - 1) % 2, (J - 1) // 2)
        for d in range(2):
            for X in range(2):
                rdmas[(d, X, MSGS - 2)].wait_send()
                rdmas[(d, X, MSGS - 1)].wait_send()
                st = pending_store.pop((d, X), None)
                if st is not None:
                    st.wait()

    return pl.pallas_call(
        body,
        out_shape=jax.ShapeDtypeStruct((M, N), jnp.float32),
        in_specs=[
            pl.BlockSpec(memory_space=pl.ANY),
            pl.BlockSpec(memory_space=pltpu.SMEM),
        ],
        out_specs=pl.BlockSpec(memory_space=pl.ANY),
        scratch_shapes=[
            pltpu.VMEM((2, 2, 2, MC, NC), jnp.float32),
            pltpu.VMEM((2, 2, 4, MC, NC), jnp.float32),
            pltpu.VMEM((2, 2, MC, NC), jnp.int32),
            pltpu.SemaphoreType.DMA((2, 2, 2)),
            pltpu.SemaphoreType.DMA((2, 2, 4)),
            pltpu.SemaphoreType.DMA((2, 2)),
            pltpu.SemaphoreType.DMA((2, 2)),
            pltpu.SemaphoreType.DMA((2, 2)),
        ],
        compiler_params=pltpu.CompilerParams(
            collective_id=0,
            vmem_limit_bytes=100 * 1024 * 1024,
        ),
    )(partial, scale)


# device time: 1180507 ns/iter; 1.2042x vs baseline; 1.0683x over previous
import jax
import jax.numpy as jnp
from jax import lax
from jax.experimental import pallas as pl
from jax.experimental.pallas import tpu as pltpu

N_DEV = 4
M, N = 4096, 8192
MC = M // N_DEV
NC = 512
HALF = N // 2
QUARTER = HALF // 2
T = QUARTER // NC
STEPS = 2 * (N_DEV - 1)
MSGS = T * STEPS


def kernel(x, w_mat, scale_x, scale_w):
    scale = (scale_x * scale_w).reshape(1, 1)

    def body(x_ref, w_ref, scale_ref, out_ref,
             send_buf, recv_buf,
             send_sems, recv_sems, store_sems, own_sems):
        my = lax.axis_index("i")
        sgn = (1, -1)
        nbo = (jnp.mod(my + 1, N_DEV), jnp.mod(my + N_DEV - 1, N_DEV))

        def cidx(d, j):
            return jnp.mod(my - sgn[d] * j + 4 * N_DEV, N_DEV)

        def col0(d, X, t):
            return d * HALF + X * QUARTER + t * NC

        def local_chunk(c, col):
            xs = x_ref[pl.ds(c * MC, MC), :]
            ws = w_ref[:, pl.ds(col, NC)]
            return lax.dot_general(
                xs, ws, (((1,), (0,)), ((), ())),
                preferred_element_type=jnp.int32,
            ).astype(jnp.float32)

        def rdma(d, X, k):
            return pltpu.make_async_remote_copy(
                src_ref=send_buf.at[d, X, k % 2],
                dst_ref=recv_buf.at[d, X, k % 4],
                send_sem=send_sems.at[d, X, k % 2],
                recv_sem=recv_sems.at[d, X, k % 4],
                device_id=(nbo[d],),
                device_id_type=pl.DeviceIdType.MESH,
            )

        barrier = pltpu.get_barrier_semaphore()
        for d in range(2):
            pl.semaphore_signal(
                barrier, inc=1, device_id=(nbo[d],),
                device_id_type=pl.DeviceIdType.MESH,
            )
        pl.semaphore_wait(barrier, 2)

        for d in range(2):
            for X in range(2):
                send_buf[d, X, 0] = local_chunk(my, col0(d, X, 0))

        rdmas = {}
        pending_store = {}
        pending_own = {}

        def process(d, X, kk):
            t, s = divmod(kk, STEPS)
            col = col0(d, X, t)
            rdmas[(d, X, kk)].wait_recv()
            if s < 2:
                c = cidx(d, s + 1)
                if kk >= 1:
                    rdmas[(d, X, kk - 1)].wait_send()
                send_buf[d, X, (kk + 1) % 2] = (
                    recv_buf[d, X, kk % 4] + local_chunk(c, col)
                )
            elif s == 2:
                c = cidx(d, 3)
                rdmas[(d, X, kk - 1)].wait_send()
                owned = (
                    recv_buf[d, X, kk % 4] + local_chunk(c, col)
                ) * scale_ref[0, 0]
                send_buf[d, X, (kk + 1) % 2] = jnp.maximum(owned, 0.0)
                ost = pltpu.make_async_copy(
                    send_buf.at[d, X, (kk + 1) % 2],
                    out_ref.at[pl.ds(c * MC, MC), pl.ds(col, NC)],
                    own_sems.at[d, X],
                )
                ost.start()
                pending_own[(d, X)] = ost
            else:
                c = cidx(d, s - 3)
                st = pltpu.make_async_copy(
                    recv_buf.at[d, X, kk % 4],
                    out_ref.at[pl.ds(c * MC, MC), pl.ds(col, NC)],
                    store_sems.at[d, X],
                )
                st.start()
                pending_store[(d, X)] = st
                if s < STEPS - 1:
                    rdmas[(d, X, kk - 1)].wait_send()
                    if s == 4:
                        pending_own.pop((d, X)).wait()
                    send_buf[d, X, (kk + 1) % 2] = recv_buf[d, X, kk % 4]
                elif t + 1 < T:
                    rdmas[(d, X, kk - 1)].wait_send()
                    send_buf[d, X, 0] = local_chunk(my, col0(d, X, t + 1))

        J = 2 * MSGS
        for j in range(J):
            X, kk = j % 2, j // 2
            for d in range(2):
                st = pending_store.pop((d, X), None)
                if st is not None:
                    st.wait()
                r = rdma(d, X, kk)
                rdmas[(d, X, kk)] = r
                r.start()
            if j >= 1:
                Xp, kp = (j - 1) % 2, (j - 1) // 2
                for d in range(2):
                    process(d, Xp, kp)
        for d in range(2):
            process(d, (J - 1) % 2, (J - 1) // 2)
        for d in range(2):
            for X in range(2):
                rdmas[(d, X, MSGS - 2)].wait_send()
                rdmas[(d, X, MSGS - 1)].wait_send()
                st = pending_store.pop((d, X), None)
                if st is not None:
                    st.wait()

    return pl.pallas_call(
        body,
        out_shape=jax.ShapeDtypeStruct((M, N), jnp.float32),
        in_specs=[
            pl.BlockSpec(memory_space=pltpu.VMEM),
            pl.BlockSpec(memory_space=pltpu.VMEM),
            pl.BlockSpec(memory_space=pltpu.SMEM),
        ],
        out_specs=pl.BlockSpec(memory_space=pl.ANY),
        scratch_shapes=[
            pltpu.VMEM((2, 2, 2, MC, NC), jnp.float32),
            pltpu.VMEM((2, 2, 4, MC, NC), jnp.float32),
            pltpu.SemaphoreType.DMA((2, 2, 2)),
            pltpu.SemaphoreType.DMA((2, 2, 4)),
            pltpu.SemaphoreType.DMA((2, 2)),
            pltpu.SemaphoreType.DMA((2, 2)),
        ],
        compiler_params=pltpu.CompilerParams(
            collective_id=0,
            vmem_limit_bytes=100 * 1024 * 1024,
        ),
    )(x, w_mat, scale)


# device time: 641219 ns/iter; 2.2169x vs baseline; 1.8410x over previous
import jax
import jax.numpy as jnp
from jax import lax
from jax.experimental import pallas as pl
from jax.experimental.pallas import tpu as pltpu

N_DEV = 4
M, N = 4096, 8192
MC = M // N_DEV
NC = 512
HALF = N // 2
QUARTER = HALF // 2
T = QUARTER // NC
STEPS = 2 * (N_DEV - 1)
MSGS = T * STEPS


def kernel(x, w_mat, scale_x, scale_w):
    scale = (scale_x * scale_w).reshape(1, 1)

    def body(x_ref, w_ref, scale_ref, out_ref,
             send_buf, recv_buf, stage,
             send_sems, recv_sems, stage_sems):
        my = lax.axis_index("i")
        sgn = (1, -1)
        nbo = (jnp.mod(my + 1, N_DEV), jnp.mod(my + N_DEV - 1, N_DEV))

        def cidx(d, j):
            return jnp.mod(my - sgn[d] * j + 4 * N_DEV, N_DEV)

        def col0(d, X, t):
            return d * HALF + X * QUARTER + t * NC

        def local_chunk(c, col):
            xs = x_ref[pl.ds(c * MC, MC), :]
            ws = w_ref[:, pl.ds(col, NC)]
            return lax.dot_general(
                xs, ws, (((1,), (0,)), ((), ())),
                preferred_element_type=jnp.int32,
            ).astype(jnp.float32)

        def rdma(d, X, k):
            return pltpu.make_async_remote_copy(
                src_ref=send_buf.at[d, X, k % 2],
                dst_ref=recv_buf.at[d, X, k % 4],
                send_sem=send_sems.at[d, X, k % 2],
                recv_sem=recv_sems.at[d, X, k % 4],
                device_id=(nbo[d],),
                device_id_type=pl.DeviceIdType.MESH,
            )

        barrier = pltpu.get_barrier_semaphore()
        for d in range(2):
            pl.semaphore_signal(
                barrier, inc=1, device_id=(nbo[d],),
                device_id_type=pl.DeviceIdType.MESH,
            )
        pl.semaphore_wait(barrier, 2)

        for d in range(2):
            for X in range(2):
                send_buf[d, X, 0] = local_chunk(
                    my, col0(d, X, 0)).astype(jnp.bfloat16)

        rdmas = {}
        stage_pending = {}
        stage_use = {(d, X): 0 for d in range(2) for X in range(2)}

        def stage_out(d, X, val_f32, c, col):
            slot = stage_use[(d, X)] % 2
            stage_use[(d, X)] += 1
            prev = stage_pending.pop((d, X, slot), None)
            if prev is not None:
                prev.wait()
            stage[d, X, slot] = val_f32
            dma = pltpu.make_async_copy(
                stage.at[d, X, slot],
                out_ref.at[pl.ds(c * MC, MC), pl.ds(col, NC)],
                stage_sems.at[d, X, slot],
            )
            dma.start()
            stage_pending[(d, X, slot)] = dma

        def process(d, X, kk):
            t, s = divmod(kk, STEPS)
            col = col0(d, X, t)
            rdmas[(d, X, kk)].wait_recv()
            if s < 2:
                c = cidx(d, s + 1)
                if kk >= 1:
                    rdmas[(d, X, kk - 1)].wait_send()
                send_buf[d, X, (kk + 1) % 2] = (
                    recv_buf[d, X, kk % 4].astype(jnp.float32)
                    + local_chunk(c, col)
                ).astype(jnp.bfloat16)
            elif s == 2:
                c = cidx(d, 3)
                rdmas[(d, X, kk - 1)].wait_send()
                owned = jnp.maximum(
                    (recv_buf[d, X, kk % 4].astype(jnp.float32)
                     + local_chunk(c, col)) * scale_ref[0, 0],
                    0.0,
                )
                send_buf[d, X, (kk + 1) % 2] = owned.astype(jnp.bfloat16)
                stage_out(d, X, owned, c, col)
            else:
                c = cidx(d, s - 3)
                stage_out(
                    d, X, recv_buf[d, X, kk % 4].astype(jnp.float32),
                    c, col,
                )
                if s < STEPS - 1:
                    rdmas[(d, X, kk - 1)].wait_send()
                    send_buf[d, X, (kk + 1) % 2] = recv_buf[d, X, kk % 4]
                elif t + 1 < T:
                    rdmas[(d, X, kk - 1)].wait_send()
                    send_buf[d, X, 0] = local_chunk(
                        my, col0(d, X, t + 1)).astype(jnp.bfloat16)

        J = 2 * MSGS
        for j in range(J):
            X, kk = j % 2, j // 2
            for d in range(2):
                r = rdma(d, X, kk)
                rdmas[(d, X, kk)] = r
                r.start()
            if j >= 1:
                Xp, kp = (j - 1) % 2, (j - 1) // 2
                for d in range(2):
                    process(d, Xp, kp)
        for d in range(2):
            process(d, (J - 1) % 2, (J - 1) // 2)
        for d in range(2):
            for X in range(2):
                rdmas[(d, X, MSGS - 2)].wait_send()
                rdmas[(d, X, MSGS - 1)].wait_send()
                for slot in range(2):
                    st = stage_pending.pop((d, X, slot), None)
                    if st is not None:
                        st.wait()

    return pl.pallas_call(
        body,
        out_shape=jax.ShapeDtypeStruct((M, N), jnp.float32),
        in_specs=[
            pl.BlockSpec(memory_space=pltpu.VMEM),
            pl.BlockSpec(memory_space=pltpu.VMEM),
            pl.BlockSpec(memory_space=pltpu.SMEM),
        ],
        out_specs=pl.BlockSpec(memory_space=pl.ANY),
        scratch_shapes=[
            pltpu.VMEM((2, 2, 2, MC, NC), jnp.bfloat16),
            pltpu.VMEM((2, 2, 4, MC, NC), jnp.bfloat16),
            pltpu.VMEM((2, 2, 2, MC, NC), jnp.float32),
            pltpu.SemaphoreType.DMA((2, 2, 2)),
            pltpu.SemaphoreType.DMA((2, 2, 4)),
            pltpu.SemaphoreType.DMA((2, 2, 2)),
        ],
        compiler_params=pltpu.CompilerParams(
            collective_id=0,
            vmem_limit_bytes=100 * 1024 * 1024,
        ),
    )(x, w_mat, scale)


# device time: 641086 ns/iter; 2.2174x vs baseline; 1.0002x over previous
import jax
import jax.numpy as jnp
from jax import lax
from jax.experimental import pallas as pl
from jax.experimental.pallas import tpu as pltpu

N_DEV = 4
M, N = 4096, 8192
MC = M // N_DEV
NC = 512
HALF = N // 2
QUARTER = HALF // 2
T = QUARTER // NC
STEPS = 2 * (N_DEV - 1)
MSGS = T * STEPS


def kernel(x, w_mat, scale_x, scale_w):
    scale = (scale_x * scale_w).reshape(1, 1)

    def body(x_ref, w_ref, scale_ref, out_ref,
             send_buf, recv_buf, stage,
             send_sems, recv_sems, stage_sems):
        my = lax.axis_index("i")
        sgn = (1, -1)
        nbo = (jnp.mod(my + 1, N_DEV), jnp.mod(my + N_DEV - 1, N_DEV))

        def cidx(d, j):
            return jnp.mod(my - sgn[d] * j + 4 * N_DEV, N_DEV)

        def col0(d, X, t):
            return d * HALF + X * QUARTER + t * NC

        def local_chunk(c, col):
            xs = x_ref[pl.ds(c * MC, MC), :]
            ws = w_ref[:, pl.ds(col, NC)]
            return lax.dot_general(
                xs, ws, (((1,), (0,)), ((), ())),
                preferred_element_type=jnp.int32,
            ).astype(jnp.float32)

        def rdma(d, X, k):
            return pltpu.make_async_remote_copy(
                src_ref=send_buf.at[d, X, k % 2],
                dst_ref=recv_buf.at[d, X, k % 4],
                send_sem=send_sems.at[d, X, k % 2],
                recv_sem=recv_sems.at[d, X, k % 4],
                device_id=(nbo[d],),
                device_id_type=pl.DeviceIdType.MESH,
            )

        for d in range(2):
            for X in range(2):
                send_buf[d, X, 0] = local_chunk(
                    my, col0(d, X, 0)).astype(jnp.bfloat16)

        barrier = pltpu.get_barrier_semaphore()
        for d in range(2):
            pl.semaphore_signal(
                barrier, inc=1, device_id=(nbo[d],),
                device_id_type=pl.DeviceIdType.MESH,
            )
        pl.semaphore_wait(barrier, 2)

        rdmas = {}
        stage_pending = {}
        stage_use = {(d, X): 0 for d in range(2) for X in range(2)}

        def stage_out(d, X, val_f32, c, col):
            slot = stage_use[(d, X)] % 2
            stage_use[(d, X)] += 1
            prev = stage_pending.pop((d, X, slot), None)
            if prev is not None:
                prev.wait()
            stage[d, X, slot] = val_f32
            dma = pltpu.make_async_copy(
                stage.at[d, X, slot],
                out_ref.at[pl.ds(c * MC, MC), pl.ds(col, NC)],
                stage_sems.at[d, X, slot],
            )
            dma.start()
            stage_pending[(d, X, slot)] = dma

        def process(d, X, kk):
            t, s = divmod(kk, STEPS)
            col = col0(d, X, t)
            rdmas[(d, X, kk)].wait_recv()
            if s < 2:
                c = cidx(d, s + 1)
                if kk >= 1:
                    rdmas[(d, X, kk - 1)].wait_send()
                send_buf[d, X, (kk + 1) % 2] = (
                    recv_buf[d, X, kk % 4].astype(jnp.float32)
                    + local_chunk(c, col)
                ).astype(jnp.bfloat16)
            elif s == 2:
                c = cidx(d, 3)
                rdmas[(d, X, kk - 1)].wait_send()
                owned = jnp.maximum(
                    (recv_buf[d, X, kk % 4].astype(jnp.float32)
                     + local_chunk(c, col)) * scale_ref[0, 0],
                    0.0,
                )
                send_buf[d, X, (kk + 1) % 2] = owned.astype(jnp.bfloat16)
                stage_out(d, X, owned, c, col)
            else:
                c = cidx(d, s - 3)
                stage_out(
                    d, X, recv_buf[d, X, kk % 4].astype(jnp.float32),
                    c, col,
                )
                if s < STEPS - 1:
                    rdmas[(d, X, kk - 1)].wait_send()
                    send_buf[d, X, (kk + 1) % 2] = recv_buf[d, X, kk % 4]
                elif t + 1 < T:
                    rdmas[(d, X, kk - 1)].wait_send()
                    send_buf[d, X, 0] = local_chunk(
                        my, col0(d, X, t + 1)).astype(jnp.bfloat16)

        J = 2 * MSGS
        for j in range(J):
            X, kk = j % 2, j // 2
            for d in range(2):
                r = rdma(d, X, kk)
                rdmas[(d, X, kk)] = r
                r.start()
            if j >= 1:
                Xp, kp = (j - 1) % 2, (j - 1) // 2
                for d in range(2):
                    process(d, Xp, kp)
        for d in range(2):
            process(d, (J - 1) % 2, (J - 1) // 2)
        for d in range(2):
            for X in range(2):
                rdmas[(d, X, MSGS - 2)].wait_send()
                rdmas[(d, X, MSGS - 1)].wait_send()
                for slot in range(2):
                    st = stage_pending.pop((d, X, slot), None)
                    if st is not None:
                        st.wait()

    return pl.pallas_call(
        body,
        out_shape=jax.ShapeDtypeStruct((M, N), jnp.float32),
        in_specs=[
            pl.BlockSpec(memory_space=pltpu.VMEM),
            pl.BlockSpec(memory_space=pltpu.VMEM),
            pl.BlockSpec(memory_space=pltpu.SMEM),
        ],
        out_specs=pl.BlockSpec(memory_space=pl.ANY),
        scratch_shapes=[
            pltpu.VMEM((2, 2, 2, MC, NC), jnp.bfloat16),
            pltpu.VMEM((2, 2, 4, MC, NC), jnp.bfloat16),
            pltpu.VMEM((2, 2, 2, MC, NC), jnp.float32),
            pltpu.SemaphoreType.DMA((2, 2, 2)),
            pltpu.SemaphoreType.DMA((2, 2, 4)),
            pltpu.SemaphoreType.DMA((2, 2, 2)),
        ],
        compiler_params=pltpu.CompilerParams(
            collective_id=0,
            vmem_limit_bytes=100 * 1024 * 1024,
        ),
    )(x, w_mat, scale)
